# Initial kernel scaffold; baseline (speedup 1.0000x reference)
import jax
import jax.numpy as jnp
from jax.experimental import pallas as pl


def kernel(x, edge_index, eps, W1, b1, W2, b2, Wfc, bfc):
    raise NotImplementedError("write your pallas kernel here")


# speedup vs baseline: 9.6942x; 9.6942x over previous
import functools

import jax
import jax.numpy as jnp
from jax import lax
from jax.experimental import pallas as pl
from jax.experimental.pallas import tpu as pltpu
from jax.experimental.pallas import tpu_sc as plsc

N_NODES = 10000
N_EDGES = 320000
D = 128

NC = 2
NS = 16
NW = NC * NS
EPW = N_EDGES // NW
K = 80
G = 5
NG = EPW // (G * K)
NP = 10240
RPS = NP // NS
ZR = 64

_mesh = plsc.VectorSubcoreMesh(
    core_axis_name="c", subcore_axis_name="s", num_cores=NC, num_subcores=NS
)


def _sc_aggregate_body(x_hbm, src_hbm, dst_hbm, out_hbm,
                       sidx_v, didx_v, rows_v, zeros_v, acc):
    c = lax.axis_index("c")
    s = lax.axis_index("s")
    wid = s * NC + c

    z16 = jnp.zeros((16,), jnp.float32)

    @pl.loop(0, ZR)
    def _fill(i):
        for j in range(D // 16):
            zeros_v[i, pl.ds(j * 16, 16)] = z16

    for t in range(RPS // ZR):
        pltpu.sync_copy(zeros_v, acc.at[pl.ds(s * RPS + t * ZR, ZR)])
    plsc.subcore_barrier()

    @pl.loop(0, NG)
    def _group(g):
        pltpu.sync_copy(src_hbm.at[wid, g], sidx_v)
        pltpu.sync_copy(dst_hbm.at[wid, g], didx_v)
        for j in range(G):
            pltpu.sync_copy(x_hbm.at[sidx_v.at[j]], rows_v)
            pltpu.sync_copy(rows_v, acc.at[didx_v.at[j]], add=True)

    plsc.subcore_barrier()
    pltpu.sync_copy(acc.at[pl.ds(s * RPS, RPS)],
                    out_hbm.at[c, pl.ds(s * RPS, RPS)])


_sc_aggregate = pl.kernel(
    _sc_aggregate_body,
    out_type=jax.ShapeDtypeStruct((NC, NP, D), jnp.float32),
    mesh=_mesh,
    scratch_types=[
        pltpu.VMEM((G, K), jnp.int32),
        pltpu.VMEM((G, K), jnp.int32),
        pltpu.VMEM((K, D), jnp.float32),
        pltpu.VMEM((ZR, D), jnp.float32),
        pltpu.MemorySpace.VMEM_SHARED((NP, D), jnp.float32),
    ],
)


def _tc_mlp_body(scale_ref, x_ref, p0_ref, p1_ref, w1_ref, b1_ref,
                 w2_ref, b2_ref, wfc_ref, bs_ref, s_ref):
    h = x_ref[...] * scale_ref[0] + p0_ref[...] + p1_ref[...]
    h = jnp.maximum(
        jnp.dot(h, w1_ref[...], preferred_element_type=jnp.float32)
        + b1_ref[...], 0.0)
    h = jnp.maximum(
        jnp.dot(h, w2_ref[...], preferred_element_type=jnp.float32)
        + b2_ref[...], 0.0)
    s_ref[...] = (
        jnp.dot(h, wfc_ref[...], preferred_element_type=jnp.float32)
        + bs_ref[...])


_RB = 1000

_tc_mlp = pl.pallas_call(
    _tc_mlp_body,
    grid=(N_NODES // _RB,),
    in_specs=[
        pl.BlockSpec(memory_space=pltpu.MemorySpace.SMEM),
        pl.BlockSpec((_RB, D), lambda i: (i, 0)),
        pl.BlockSpec((_RB, D), lambda i: (i, 0)),
        pl.BlockSpec((_RB, D), lambda i: (i, 0)),
        pl.BlockSpec((D, D), lambda i: (0, 0)),
        pl.BlockSpec((1, D), lambda i: (0, 0)),
        pl.BlockSpec((D, D), lambda i: (0, 0)),
        pl.BlockSpec((1, D), lambda i: (0, 0)),
        pl.BlockSpec((D, 8), lambda i: (0, 0)),
        pl.BlockSpec((1, 8), lambda i: (0, 0)),
    ],
    out_specs=pl.BlockSpec((_RB, 8), lambda i: (i, 0)),
    out_shape=jax.ShapeDtypeStruct((N_NODES, 8), jnp.float32),
)


def _sc_scores_body(s1_hbm, s2_hbm, src_hbm, dst_hbm, out_hbm,
                    sidx_v, didx_v, out_v, s1_sh, s2_sh):
    c = lax.axis_index("c")
    s = lax.axis_index("s")
    wid = s * NC + c

    @pl.when(s == 0)
    def _stage():
        pltpu.sync_copy(s1_hbm, s1_sh)
        pltpu.sync_copy(s2_hbm, s2_sh)

    plsc.subcore_barrier()

    @pl.loop(0, NG)
    def _group(g):
        pltpu.sync_copy(src_hbm.at[wid, g], sidx_v)
        pltpu.sync_copy(dst_hbm.at[wid, g], didx_v)
        for j in range(G):
            chunk = out_v.at[pl.ds(g * G * K + j * K, K)]
            pltpu.sync_copy(s1_sh.at[sidx_v.at[j]], chunk)
            pltpu.sync_copy(s2_sh.at[didx_v.at[j]], chunk, add=True)

    pltpu.sync_copy(out_v, out_hbm.at[pl.ds(wid * EPW, EPW)])


_sc_scores = pl.kernel(
    _sc_scores_body,
    out_type=jax.ShapeDtypeStruct((N_EDGES,), jnp.float32),
    mesh=_mesh,
    scratch_types=[
        pltpu.VMEM((G, K), jnp.int32),
        pltpu.VMEM((G, K), jnp.int32),
        pltpu.VMEM((EPW,), jnp.float32),
        pltpu.MemorySpace.VMEM_SHARED((N_NODES,), jnp.float32),
        pltpu.MemorySpace.VMEM_SHARED((N_NODES,), jnp.float32),
    ],
)


def kernel(x, edge_index, eps, W1, b1, W2, b2, Wfc, bfc):
    src = edge_index[0].astype(jnp.int32)
    dst = edge_index[1].astype(jnp.int32)
    src3 = src.reshape(NW, NG, G, K)
    dst3 = dst.reshape(NW, NG, G, K)

    parts = _sc_aggregate(x, src3, dst3)

    scale = (1.0 + eps).reshape(1).astype(jnp.float32)
    wfc2 = jnp.pad(Wfc.reshape(2, D).T, ((0, 0), (0, 6)))
    bs = jnp.zeros((1, 8), jnp.float32).at[0, 0].set(bfc[0])
    S = _tc_mlp(scale, x, parts[0], parts[1], W1, b1.reshape(1, D),
                W2, b2.reshape(1, D), wfc2, bs)

    return _sc_scores(S[:, 0], S[:, 1], src3, dst3)
